# baseline (speedup 1.0000x reference)
import jax
import jax.numpy as jnp
from jax import lax
from jax.experimental import pallas as pl
from jax.experimental.pallas import tpu as pltpu
from jax.experimental.pallas import tpu_sc as plsc

_INFO = plsc.get_sparse_core_info()
_NC = _INFO.num_cores
_NS = _INFO.num_subcores
_NW = _NC * _NS

CHARSET = 1000
B, L = 4096, 20
_BPW = B // _NW
_C = L
_NCHUNK = _BPW
_NBUF = 4
_T = _NCHUNK // _NBUF
_REM = _NCHUNK - (_T - 1) * _NBUF


def _gather_body(table_hbm, idx_hbm, out_hbm, idx_v,
                 r0, r1, r2, r3, g0, g1, g2, g3, w0, w1, w2, w3):
    bufs = [r0, r1, r2, r3]
    gs = [g0, g1, g2, g3]
    ws = [w0, w1, w2, w3]
    wid = lax.axis_index("s") * _NC + lax.axis_index("c")
    base_b = wid * _BPW
    pltpu.sync_copy(idx_hbm.at[wid], idx_v)

    def start_gather(j, b):
        pltpu.async_copy(table_hbm.at[idx_v.at[j]], bufs[b], gs[b])

    def wait_gather(j, b):
        pltpu.make_async_copy(table_hbm.at[idx_v.at[j]], bufs[b], gs[b]).wait()

    def start_write(j, b):
        pltpu.async_copy(bufs[b], out_hbm.at[base_b + j], ws[b])

    def wait_write(j, b):
        pltpu.make_async_copy(bufs[b], out_hbm.at[base_b + j], ws[b]).wait()

    for b in range(_NBUF):
        start_gather(b, b)

    def outer(g, carry):
        jj = g * _NBUF
        for b in range(_NBUF):
            wait_gather(jj + b, b)
            start_write(jj + b, b)
        for b in range(_NBUF):
            wait_write(jj + b, b)
            start_gather(jj + _NBUF + b, b)
        return carry

    lax.fori_loop(0, _T - 1, outer, 0)

    jj = (_T - 1) * _NBUF
    for k in range(_REM):
        b = k % _NBUF
        wait_gather(jj + k, b)
        start_write(jj + k, b)
    for k in range(_REM):
        b = k % _NBUF
        wait_write(jj + k, b)


@jax.jit
def _run(table, idx3):
    mesh = plsc.VectorSubcoreMesh(core_axis_name="c", subcore_axis_name="s")
    f = pl.kernel(
        _gather_body,
        out_type=jax.ShapeDtypeStruct((B, L, CHARSET), jnp.float32),
        mesh=mesh,
        scratch_types=(
            [pltpu.VMEM((_NCHUNK, _C), jnp.int32)]
            + [pltpu.VMEM((L, CHARSET), jnp.float32) for _ in range(_NBUF)]
            + [pltpu.SemaphoreType.DMA for _ in range(2 * _NBUF)]
        ),
        compiler_params=pltpu.CompilerParams(use_tc_tiling_on_sc=False),
    )
    return f(table, idx3)


def kernel(index, targets, embedding_table):
    idx3 = index.astype(jnp.int32).reshape(_NW, _NCHUNK, _C)
    return _run(embedding_table, idx3)
